# baseline (speedup 1.0000x reference)
import functools

import jax
import jax.numpy as jnp
from jax import lax
from jax.experimental import pallas as pl
from jax.experimental.pallas import tpu as pltpu
from jax.experimental.pallas import tpu_sc as plsc

_N = 8192
_D = 64
_K = 8192
_TM = 512
_KC = 2048
_GT = _N // _TM
_KS = _K // _KC

_NW = 32
_BPW = _N // _NW
_CH = 128


def _argmin_body(x_ref, e_ref, xsq_ref, esq_ref, idx_ref, dsum_ref, best_ref):
    j = pl.program_id(1)
    xb = (x_ref[...] * 2.0).astype(jnp.bfloat16)
    e = e_ref[...]
    mm = lax.dot_general(xb, e, (((1,), (1,)), ((), ())),
                         preferred_element_type=jnp.float32)
    d = (xsq_ref[...] + esq_ref[...]) - mm
    m = jnp.min(d, axis=1, keepdims=True)
    ii = lax.broadcasted_iota(jnp.int32, (_TM, _KC), 1) + j * _KC
    cand = jnp.min(jnp.where(d == m, ii, _K), axis=1, keepdims=True)

    @pl.when(j == 0)
    def _():
        best_ref[...] = m.astype(jnp.bfloat16).astype(jnp.float32)
        idx_ref[...] = cand

    @pl.when(j > 0)
    def _():
        better = m < best_ref[...]
        idx_ref[...] = jnp.where(better, cand, idx_ref[...])
        best_ref[...] = jnp.where(better, m, best_ref[...]).astype(
            jnp.bfloat16).astype(jnp.float32)

    @pl.when(j == _KS - 1)
    def _():
        dsum_ref[...] = jnp.sum(best_ref[...])[None, None, None]


_argmin_call = pl.pallas_call(
    _argmin_body,
    grid=(_GT, _KS),
    in_specs=[
        pl.BlockSpec((_TM, _D), lambda i, j: (i, 0)),
        pl.BlockSpec((_KC, _D), lambda i, j: (j, 0)),
        pl.BlockSpec((_TM, 1), lambda i, j: (i, 0)),
        pl.BlockSpec((1, _KC), lambda i, j: (0, j)),
    ],
    out_specs=[
        pl.BlockSpec((_TM, 1), lambda i, j: (i, 0)),
        pl.BlockSpec((1, 1, 1), lambda i, j: (i, 0, 0)),
    ],
    out_shape=[
        jax.ShapeDtypeStruct((_N, 1), jnp.int32),
        jax.ShapeDtypeStruct((_GT, 1, 1), jnp.float32),
    ],
    scratch_shapes=[pltpu.VMEM((_TM, 1), jnp.float32)],
    compiler_params=pltpu.CompilerParams(
        dimension_semantics=("arbitrary", "arbitrary")),
)


_DP = 128


@functools.cache
def _make_sc_gather():
    mesh = plsc.VectorSubcoreMesh(core_axis_name="c", subcore_axis_name="s")

    @functools.partial(
        pl.kernel,
        mesh=mesh,
        out_type=jax.ShapeDtypeStruct((_N, _DP), jnp.float32),
        scratch_types=[
            pltpu.VMEM((2, _CH), jnp.int32),
            pltpu.VMEM((_BPW, _DP), jnp.float32),
            pltpu.SemaphoreType.DMA,
        ],
    )
    def gather_k(emb_hbm, idx_hbm, out_hbm, idx_v, rows_v, sem):
        wid = lax.axis_index("s") * 2 + lax.axis_index("c")
        base = wid * _BPW
        pltpu.sync_copy(idx_hbm.at[pl.ds(base, _CH)], idx_v.at[0])
        pltpu.sync_copy(idx_hbm.at[pl.ds(base + _CH, _CH)], idx_v.at[1])
        c0 = pltpu.async_copy(emb_hbm.at[idx_v.at[0]],
                              rows_v.at[pl.ds(0, _CH)], sem)
        c1 = pltpu.async_copy(emb_hbm.at[idx_v.at[1]],
                              rows_v.at[pl.ds(_CH, _CH)], sem)
        c0.wait()
        c1.wait()
        pltpu.sync_copy(rows_v, out_hbm.at[pl.ds(base, _BPW)])

    return gather_k


def kernel(inputs, emb):
    B, T, D = inputs.shape
    flat = inputs.reshape(-1, D)
    xsq = jnp.sum(flat ** 2, axis=1, keepdims=True)
    esq = jnp.sum(emb ** 2, axis=1)[None, :]
    idx2, dsum = _argmin_call(flat, emb, xsq, esq)
    idx = idx2.reshape(_N)
    emb_p = jnp.pad(emb, ((0, 0), (0, _DP - _D)))
    quantized = _make_sc_gather()(emb_p, idx)[:, :_D].reshape(B, T, D)
    e_latent_loss = jnp.sum(dsum) / jnp.float32(_N * _D)
    vq_loss = jnp.float32(0.25) * e_latent_loss
    quantized_st = inputs + (quantized - inputs)
    return (quantized_st, idx2.reshape(B, T), vq_loss, e_latent_loss,
            jnp.float32(0.0))


# speedup vs baseline: 1.2450x; 1.2450x over previous
import functools

import jax
import jax.numpy as jnp
from jax import lax
from jax.experimental import pallas as pl
from jax.experimental.pallas import tpu as pltpu
from jax.experimental.pallas import tpu_sc as plsc

_N = 8192
_D = 64
_K = 8192
_TM = 512
_KC = 2048
_GT = _N // _TM
_KS = _K // _KC

_NW = 32
_BPW = _N // _NW
_CH = 128


def _argmin_body(x_ref, e_ref, xsq_ref, esq_ref, idx_ref, dsum_ref, best_ref):
    j = pl.program_id(1)
    xb = (x_ref[...] * 2.0).astype(jnp.bfloat16)
    e = e_ref[...]
    mm = lax.dot_general(xb, e, (((1,), (1,)), ((), ())),
                         preferred_element_type=jnp.float32)
    xsq = xsq_ref[...]
    esq = esq_ref[...]
    L = 128
    bval = (xsq + esq[:, 0:L]) - mm[:, 0:L]
    bcol = jnp.zeros((_TM, L), jnp.int32)
    for c in range(1, _KC // L):
        dcol = (xsq + esq[:, c * L:(c + 1) * L]) - mm[:, c * L:(c + 1) * L]
        better = dcol < bval
        bval = jnp.where(better, dcol, bval)
        bcol = jnp.where(better, jnp.int32(c), bcol)
    m = jnp.min(bval, axis=1, keepdims=True)
    kfull = bcol * L + lax.broadcasted_iota(jnp.int32, (_TM, L), 1)
    cand = jnp.min(jnp.where(bval == m, kfull, _K), axis=1,
                   keepdims=True) + j * _KC

    @pl.when(j == 0)
    def _():
        best_ref[...] = m.astype(jnp.bfloat16).astype(jnp.float32)
        idx_ref[...] = cand

    @pl.when(j > 0)
    def _():
        better = m < best_ref[...]
        idx_ref[...] = jnp.where(better, cand, idx_ref[...])
        best_ref[...] = jnp.where(better, m, best_ref[...]).astype(
            jnp.bfloat16).astype(jnp.float32)

    @pl.when(j == _KS - 1)
    def _():
        dsum_ref[...] = jnp.sum(best_ref[...])[None, None, None]


_argmin_call = pl.pallas_call(
    _argmin_body,
    grid=(_GT, _KS),
    in_specs=[
        pl.BlockSpec((_TM, _D), lambda i, j: (i, 0)),
        pl.BlockSpec((_KC, _D), lambda i, j: (j, 0)),
        pl.BlockSpec((_TM, 1), lambda i, j: (i, 0)),
        pl.BlockSpec((1, _KC), lambda i, j: (0, j)),
    ],
    out_specs=[
        pl.BlockSpec((_TM, 1), lambda i, j: (i, 0)),
        pl.BlockSpec((1, 1, 1), lambda i, j: (i, 0, 0)),
    ],
    out_shape=[
        jax.ShapeDtypeStruct((_N, 1), jnp.int32),
        jax.ShapeDtypeStruct((_GT, 1, 1), jnp.float32),
    ],
    scratch_shapes=[pltpu.VMEM((_TM, 1), jnp.float32)],
    compiler_params=pltpu.CompilerParams(
        dimension_semantics=("arbitrary", "arbitrary")),
)


_DP = 128


@functools.cache
def _make_sc_gather():
    mesh = plsc.VectorSubcoreMesh(core_axis_name="c", subcore_axis_name="s")

    @functools.partial(
        pl.kernel,
        mesh=mesh,
        out_type=jax.ShapeDtypeStruct((_N, _DP), jnp.float32),
        scratch_types=[
            pltpu.VMEM((2, _CH), jnp.int32),
            pltpu.VMEM((_BPW, _DP), jnp.float32),
            pltpu.SemaphoreType.DMA,
        ],
    )
    def gather_k(emb_hbm, idx_hbm, out_hbm, idx_v, rows_v, sem):
        wid = lax.axis_index("s") * 2 + lax.axis_index("c")
        base = wid * _BPW
        pltpu.sync_copy(idx_hbm.at[pl.ds(base, _CH)], idx_v.at[0])
        pltpu.sync_copy(idx_hbm.at[pl.ds(base + _CH, _CH)], idx_v.at[1])
        c0 = pltpu.async_copy(emb_hbm.at[idx_v.at[0]],
                              rows_v.at[pl.ds(0, _CH)], sem)
        c1 = pltpu.async_copy(emb_hbm.at[idx_v.at[1]],
                              rows_v.at[pl.ds(_CH, _CH)], sem)
        c0.wait()
        c1.wait()
        pltpu.sync_copy(rows_v, out_hbm.at[pl.ds(base, _BPW)])

    return gather_k


def kernel(inputs, emb):
    B, T, D = inputs.shape
    flat = inputs.reshape(-1, D)
    xsq = jnp.sum(flat ** 2, axis=1, keepdims=True)
    esq = jnp.sum(emb ** 2, axis=1)[None, :]
    idx2, dsum = _argmin_call(flat, emb, xsq, esq)
    idx = idx2.reshape(_N)
    emb_p = jnp.pad(emb, ((0, 0), (0, _DP - _D)))
    quantized = _make_sc_gather()(emb_p, idx)[:, :_D].reshape(B, T, D)
    e_latent_loss = jnp.sum(dsum) / jnp.float32(_N * _D)
    vq_loss = jnp.float32(0.25) * e_latent_loss
    quantized_st = inputs + (quantized - inputs)
    return (quantized_st, idx2.reshape(B, T), vq_loss, e_latent_loss,
            jnp.float32(0.0))


# speedup vs baseline: 1.2556x; 1.0086x over previous
import functools

import jax
import jax.numpy as jnp
from jax import lax
from jax.experimental import pallas as pl
from jax.experimental.pallas import tpu as pltpu
from jax.experimental.pallas import tpu_sc as plsc

_N = 8192
_D = 64
_K = 8192
_TM = 512
_KC = 2048
_GT = _N // _TM
_KS = _K // _KC

_NW = 32
_BPW = _N // _NW
_CH = 128


def _argmin_body(x_ref, e_ref, xsq_ref, esq_ref, idx_ref, dsum_ref, best_ref):
    j = pl.program_id(1)
    xb = (x_ref[...] * 2.0).astype(jnp.bfloat16)
    e = e_ref[...]
    mm = lax.dot_general(xb, e, (((1,), (1,)), ((), ())),
                         preferred_element_type=jnp.float32)
    xsq = xsq_ref[...]
    esq = esq_ref[...]
    L = 128
    bval = (xsq + esq[:, 0:L]) - mm[:, 0:L]
    bcol = jnp.zeros((_TM, L), jnp.int32)
    for c in range(1, _KC // L):
        dcol = (xsq + esq[:, c * L:(c + 1) * L]) - mm[:, c * L:(c + 1) * L]
        better = dcol < bval
        bval = jnp.where(better, dcol, bval)
        bcol = jnp.where(better, jnp.int32(c), bcol)
    m = jnp.min(bval, axis=1, keepdims=True)
    kfull = bcol * L + lax.broadcasted_iota(jnp.int32, (_TM, L), 1)
    cand = jnp.min(jnp.where(bval == m, kfull, _K), axis=1,
                   keepdims=True) + j * _KC

    @pl.when(j == 0)
    def _():
        best_ref[...] = m.astype(jnp.bfloat16).astype(jnp.float32)
        idx_ref[...] = cand

    @pl.when(j > 0)
    def _():
        better = m < best_ref[...]
        idx_ref[...] = jnp.where(better, cand, idx_ref[...])
        best_ref[...] = jnp.where(better, m, best_ref[...]).astype(
            jnp.bfloat16).astype(jnp.float32)

    @pl.when(j == _KS - 1)
    def _():
        dsum_ref[...] = jnp.sum(best_ref[...])[None, None, None]


_argmin_call = pl.pallas_call(
    _argmin_body,
    grid=(_GT, _KS),
    in_specs=[
        pl.BlockSpec((_TM, _D), lambda i, j: (i, 0)),
        pl.BlockSpec((_KC, _D), lambda i, j: (j, 0)),
        pl.BlockSpec((_TM, 1), lambda i, j: (i, 0)),
        pl.BlockSpec((1, _KC), lambda i, j: (0, j)),
    ],
    out_specs=[
        pl.BlockSpec((_TM, 1), lambda i, j: (i, 0)),
        pl.BlockSpec((1, 1, 1), lambda i, j: (i, 0, 0)),
    ],
    out_shape=[
        jax.ShapeDtypeStruct((_N, 1), jnp.int32),
        jax.ShapeDtypeStruct((_GT, 1, 1), jnp.float32),
    ],
    scratch_shapes=[pltpu.VMEM((_TM, 1), jnp.float32)],
    compiler_params=pltpu.CompilerParams(
        dimension_semantics=("arbitrary", "arbitrary")),
)


_DP = 128


@functools.cache
def _make_sc_gather():
    mesh = plsc.VectorSubcoreMesh(core_axis_name="c", subcore_axis_name="s")

    @functools.partial(
        pl.kernel,
        mesh=mesh,
        out_type=jax.ShapeDtypeStruct((_N, _DP), jnp.float32),
        scratch_types=[
            pltpu.VMEM((2, _CH), jnp.int32),
            pltpu.VMEM((_BPW, _DP), jnp.float32),
            pltpu.SemaphoreType.DMA,
        ],
    )
    def gather_k(emb_hbm, idx_hbm, out_hbm, idx_v, rows_v, sem):
        wid = lax.axis_index("s") * 2 + lax.axis_index("c")
        base = wid * _BPW
        pltpu.sync_copy(idx_hbm.at[pl.ds(base, _CH)], idx_v.at[0])
        pltpu.sync_copy(idx_hbm.at[pl.ds(base + _CH, _CH)], idx_v.at[1])
        c0 = pltpu.async_copy(emb_hbm.at[idx_v.at[0]],
                              rows_v.at[pl.ds(0, _CH)], sem)
        c1 = pltpu.async_copy(emb_hbm.at[idx_v.at[1]],
                              rows_v.at[pl.ds(_CH, _CH)], sem)
        c0.wait()
        c1.wait()
        pltpu.sync_copy(rows_v, out_hbm.at[pl.ds(base, _BPW)])

    return gather_k


def kernel(inputs, emb):
    B, T, D = inputs.shape
    flat = inputs.reshape(-1, D)
    xsq = jnp.sum(flat ** 2, axis=1, keepdims=True)
    esq = jnp.sum(emb ** 2, axis=1)[None, :]
    idx2, dsum = _argmin_call(flat, emb, xsq, esq)
    idx = idx2.reshape(_N)
    emb_p = jnp.pad(emb, ((0, 0), (0, _DP - _D)))
    quantized = _make_sc_gather()(emb_p, idx)[:, :_D].reshape(B, T, D)
    e_latent_loss = jnp.sum(dsum) / jnp.float32(_N * _D)
    vq_loss = jnp.float32(0.25) * e_latent_loss
    return (quantized, idx2.reshape(B, T), vq_loss, e_latent_loss,
            jnp.float32(0.0))


# speedup vs baseline: 1.4903x; 1.1869x over previous
import functools

import jax
import jax.numpy as jnp
from jax import lax
from jax.experimental import pallas as pl
from jax.experimental.pallas import tpu as pltpu
from jax.experimental.pallas import tpu_sc as plsc

_N = 8192
_D = 64
_K = 8192
_TM = 512
_KC = 2048
_GT = _N // _TM
_KS = _K // _KC

_NW = 32
_BPW = _N // _NW
_CH = 128


def _argmin_body(x_ref, e_ref, xsq_ref, idx_ref, dsum_ref, best_ref):
    j = pl.program_id(1)
    xb = (x_ref[0] * 2.0).astype(jnp.bfloat16)
    e = e_ref[...]
    esq = jnp.sum(e * e, axis=1, keepdims=True)
    mm = lax.dot_general(e, xb, (((1,), (0,)), ((), ())),
                         preferred_element_type=jnp.float32)
    xsq = xsq_ref[0]
    H = 64
    bval = (xsq + esq[0:H]) - mm[0:H, :]
    bslab = jnp.zeros((H, _TM), jnp.int32)
    for c in range(1, _KC // H):
        d = (xsq + esq[c * H:(c + 1) * H]) - mm[c * H:(c + 1) * H, :]
        better = d < bval
        bval = jnp.where(better, d, bval)
        bslab = jnp.where(better, jnp.int32(c), bslab)
    m = jnp.min(bval, axis=0, keepdims=True)
    kfull = bslab * H + lax.broadcasted_iota(jnp.int32, (H, _TM), 0)
    cand = jnp.min(jnp.where(bval == m, kfull, _K), axis=0,
                   keepdims=True) + j * _KC

    @pl.when(j == 0)
    def _():
        best_ref[...] = m.astype(jnp.bfloat16).astype(jnp.float32)
        idx_ref[...] = cand[None]

    @pl.when(j > 0)
    def _():
        better = m < best_ref[...]
        idx_ref[...] = jnp.where(better, cand[None], idx_ref[...])
        best_ref[...] = jnp.where(better, m, best_ref[...]).astype(
            jnp.bfloat16).astype(jnp.float32)

    @pl.when(j == _KS - 1)
    def _():
        dsum_ref[...] = jnp.sum(best_ref[...])[None, None, None]


_argmin_call = pl.pallas_call(
    _argmin_body,
    grid=(_GT, _KS),
    in_specs=[
        pl.BlockSpec((1, _D, _TM), lambda i, j: (i, 0, 0)),
        pl.BlockSpec((_KC, _D), lambda i, j: (j, 0)),
        pl.BlockSpec((1, 1, _TM), lambda i, j: (i, 0, 0)),
    ],
    out_specs=[
        pl.BlockSpec((1, 1, _TM), lambda i, j: (i, 0, 0)),
        pl.BlockSpec((1, 1, 1), lambda i, j: (i, 0, 0)),
    ],
    out_shape=[
        jax.ShapeDtypeStruct((_GT, 1, _TM), jnp.int32),
        jax.ShapeDtypeStruct((_GT, 1, 1), jnp.float32),
    ],
    scratch_shapes=[pltpu.VMEM((1, _TM), jnp.float32)],
    compiler_params=pltpu.CompilerParams(
        dimension_semantics=("arbitrary", "arbitrary")),
)


_DP = 128


@functools.cache
def _make_sc_gather():
    mesh = plsc.VectorSubcoreMesh(core_axis_name="c", subcore_axis_name="s")

    @functools.partial(
        pl.kernel,
        mesh=mesh,
        out_type=jax.ShapeDtypeStruct((_N, _DP), jnp.float32),
        scratch_types=[
            pltpu.VMEM((2, _CH), jnp.int32),
            pltpu.VMEM((_BPW, _DP), jnp.float32),
            pltpu.SemaphoreType.DMA,
        ],
    )
    def gather_k(emb_hbm, idx_hbm, out_hbm, idx_v, rows_v, sem):
        wid = lax.axis_index("s") * 2 + lax.axis_index("c")
        base = wid * _BPW
        pltpu.sync_copy(idx_hbm.at[pl.ds(base, _CH)], idx_v.at[0])
        pltpu.sync_copy(idx_hbm.at[pl.ds(base + _CH, _CH)], idx_v.at[1])
        c0 = pltpu.async_copy(emb_hbm.at[idx_v.at[0]],
                              rows_v.at[pl.ds(0, _CH)], sem)
        c1 = pltpu.async_copy(emb_hbm.at[idx_v.at[1]],
                              rows_v.at[pl.ds(_CH, _CH)], sem)
        c0.wait()
        c1.wait()
        pltpu.sync_copy(rows_v, out_hbm.at[pl.ds(base, _BPW)])

    return gather_k


def kernel(inputs, emb):
    B, T, D = inputs.shape
    xt = jnp.swapaxes(inputs, 1, 2)
    xsq = jnp.sum(inputs ** 2, axis=2)[:, None, :]
    idx2, dsum = _argmin_call(xt, emb, xsq)
    idx = idx2.reshape(_N)
    emb_p = jnp.pad(emb, ((0, 0), (0, _DP - _D)))
    quantized = _make_sc_gather()(emb_p, idx)[:, :_D].reshape(B, T, D)
    e_latent_loss = jnp.sum(dsum) / jnp.float32(_N * _D)
    vq_loss = jnp.float32(0.25) * e_latent_loss
    return (quantized, idx2.reshape(B, T), vq_loss, e_latent_loss,
            jnp.float32(0.0))


# speedup vs baseline: 1.6393x; 1.1000x over previous
import functools

import jax
import jax.numpy as jnp
from jax import lax
from jax.experimental import pallas as pl
from jax.experimental.pallas import tpu as pltpu
from jax.experimental.pallas import tpu_sc as plsc

_N = 8192
_D = 64
_K = 8192
_TM = 512
_KC = 2048
_GT = _N // _TM
_KS = _K // _KC

_NW = 32
_BPW = _N // _NW
_CH = 128


def _argmin_body(x_ref, e_ref, xsq_ref, idx_ref, dsum_ref):
    xb = (x_ref[0] * 2.0).astype(jnp.bfloat16)
    xsq = xsq_ref[0]
    H = 64
    gbest = None
    gidx = None
    for j in range(_KS):
        e = e_ref[pl.ds(j * _KC, _KC), :]
        esq = jnp.sum(e * e, axis=1, keepdims=True)
        mm = lax.dot_general(e, xb, (((1,), (0,)), ((), ())),
                             preferred_element_type=jnp.float32)
        bval = (xsq + esq[0:H]) - mm[0:H, :]
        bslab = jnp.zeros((H, _TM), jnp.int32)
        for c in range(1, _KC // H):
            d = (xsq + esq[c * H:(c + 1) * H]) - mm[c * H:(c + 1) * H, :]
            better = d < bval
            bval = jnp.where(better, d, bval)
            bslab = jnp.where(better, jnp.int32(c), bslab)
        m = jnp.min(bval, axis=0, keepdims=True)
        kfull = bslab * H + lax.broadcasted_iota(jnp.int32, (H, _TM), 0)
        cand = jnp.min(jnp.where(bval == m, kfull, _K), axis=0,
                       keepdims=True) + j * _KC
        if j == 0:
            gbest = m.astype(jnp.bfloat16).astype(jnp.float32)
            gidx = cand
        else:
            better = m < gbest
            gidx = jnp.where(better, cand, gidx)
            gbest = jnp.where(better, m, gbest).astype(
                jnp.bfloat16).astype(jnp.float32)
    idx_ref[...] = gidx[None]
    dsum_ref[...] = jnp.sum(gbest)[None, None, None]


_argmin_call = pl.pallas_call(
    _argmin_body,
    grid=(_GT,),
    in_specs=[
        pl.BlockSpec((1, _D, _TM), lambda i: (i, 0, 0)),
        pl.BlockSpec((_K, _D), lambda i: (0, 0)),
        pl.BlockSpec((1, 1, _TM), lambda i: (i, 0, 0)),
    ],
    out_specs=[
        pl.BlockSpec((1, 1, _TM), lambda i: (i, 0, 0)),
        pl.BlockSpec((1, 1, 1), lambda i: (i, 0, 0)),
    ],
    out_shape=[
        jax.ShapeDtypeStruct((_GT, 1, _TM), jnp.int32),
        jax.ShapeDtypeStruct((_GT, 1, 1), jnp.float32),
    ],
    compiler_params=pltpu.CompilerParams(
        dimension_semantics=("arbitrary",)),
)


_DP = 128


@functools.cache
def _make_sc_gather():
    mesh = plsc.VectorSubcoreMesh(core_axis_name="c", subcore_axis_name="s")

    @functools.partial(
        pl.kernel,
        mesh=mesh,
        out_type=jax.ShapeDtypeStruct((_N, _DP), jnp.float32),
        scratch_types=[
            pltpu.VMEM((2, _CH), jnp.int32),
            pltpu.VMEM((_BPW, _DP), jnp.float32),
            pltpu.SemaphoreType.DMA,
        ],
    )
    def gather_k(emb_hbm, idx_hbm, out_hbm, idx_v, rows_v, sem):
        wid = lax.axis_index("s") * 2 + lax.axis_index("c")
        base = wid * _BPW
        pltpu.sync_copy(idx_hbm.at[pl.ds(base, _CH)], idx_v.at[0])
        pltpu.sync_copy(idx_hbm.at[pl.ds(base + _CH, _CH)], idx_v.at[1])
        c0 = pltpu.async_copy(emb_hbm.at[idx_v.at[0]],
                              rows_v.at[pl.ds(0, _CH)], sem)
        c1 = pltpu.async_copy(emb_hbm.at[idx_v.at[1]],
                              rows_v.at[pl.ds(_CH, _CH)], sem)
        c0.wait()
        c1.wait()
        pltpu.sync_copy(rows_v, out_hbm.at[pl.ds(base, _BPW)])

    return gather_k


def kernel(inputs, emb):
    B, T, D = inputs.shape
    xt = jnp.swapaxes(inputs, 1, 2)
    xsq = jnp.sum(inputs ** 2, axis=2)[:, None, :]
    idx2, dsum = _argmin_call(xt, emb, xsq)
    idx = idx2.reshape(_N)
    emb_p = jnp.pad(emb, ((0, 0), (0, _DP - _D)))
    quantized = _make_sc_gather()(emb_p, idx)[:, :_D].reshape(B, T, D)
    e_latent_loss = jnp.sum(dsum) / jnp.float32(_N * _D)
    vq_loss = jnp.float32(0.25) * e_latent_loss
    return (quantized, idx2.reshape(B, T), vq_loss, e_latent_loss,
            jnp.float32(0.0))


# speedup vs baseline: 1.8726x; 1.1423x over previous
import functools

import jax
import jax.numpy as jnp
from jax import lax
from jax.experimental import pallas as pl
from jax.experimental.pallas import tpu as pltpu
from jax.experimental.pallas import tpu_sc as plsc

_N = 8192
_D = 64
_K = 8192
_TM = 512
_KC = 2048
_GT = _N // _TM
_KS = _K // _KC

_NW = 32
_BPW = _N // _NW
_CH = 128


def _argmin_body(x_ref, e_ref, xsq_ref, idx_ref, dsum_ref):
    xb = (x_ref[0] * 2.0).astype(jnp.bfloat16)
    xsq = xsq_ref[0]
    H = 16
    gbest = None
    gidx = None
    for j in range(_KS):
        e = e_ref[pl.ds(j * _KC, _KC), :]
        esq = jnp.sum(e * e, axis=1, keepdims=True)
        mm = lax.dot_general(e, xb, (((1,), (0,)), ((), ())),
                             preferred_element_type=jnp.float32)
        bval = (xsq + esq[0:H]) - mm[0:H, :]
        bslab = jnp.zeros((H, _TM), jnp.int32)
        for c in range(1, _KC // H):
            d = (xsq + esq[c * H:(c + 1) * H]) - mm[c * H:(c + 1) * H, :]
            better = d < bval
            bval = jnp.where(better, d, bval)
            bslab = jnp.where(better, jnp.int32(c), bslab)
        m = jnp.min(bval, axis=0, keepdims=True)
        kfull = bslab * H + lax.broadcasted_iota(jnp.int32, (H, _TM), 0)
        cand = jnp.min(jnp.where(bval == m, kfull, _K), axis=0,
                       keepdims=True) + j * _KC
        if j == 0:
            gbest = m.astype(jnp.bfloat16).astype(jnp.float32)
            gidx = cand
        else:
            better = m < gbest
            gidx = jnp.where(better, cand, gidx)
            gbest = jnp.where(better, m, gbest).astype(
                jnp.bfloat16).astype(jnp.float32)
    idx_ref[...] = gidx[None]
    dsum_ref[...] = jnp.sum(gbest)[None, None, None]


_argmin_call = pl.pallas_call(
    _argmin_body,
    grid=(_GT,),
    in_specs=[
        pl.BlockSpec((1, _D, _TM), lambda i: (i, 0, 0)),
        pl.BlockSpec((_K, _D), lambda i: (0, 0)),
        pl.BlockSpec((1, 1, _TM), lambda i: (i, 0, 0)),
    ],
    out_specs=[
        pl.BlockSpec((1, 1, _TM), lambda i: (i, 0, 0)),
        pl.BlockSpec((1, 1, 1), lambda i: (i, 0, 0)),
    ],
    out_shape=[
        jax.ShapeDtypeStruct((_GT, 1, _TM), jnp.int32),
        jax.ShapeDtypeStruct((_GT, 1, 1), jnp.float32),
    ],
    compiler_params=pltpu.CompilerParams(
        dimension_semantics=("arbitrary",)),
)


_DP = 128


@functools.cache
def _make_sc_gather():
    mesh = plsc.VectorSubcoreMesh(core_axis_name="c", subcore_axis_name="s")

    @functools.partial(
        pl.kernel,
        mesh=mesh,
        out_type=jax.ShapeDtypeStruct((_N, _DP), jnp.float32),
        scratch_types=[
            pltpu.VMEM((2, _CH), jnp.int32),
            pltpu.VMEM((_BPW, _DP), jnp.float32),
            pltpu.SemaphoreType.DMA,
        ],
    )
    def gather_k(emb_hbm, idx_hbm, out_hbm, idx_v, rows_v, sem):
        wid = lax.axis_index("s") * 2 + lax.axis_index("c")
        base = wid * _BPW
        pltpu.sync_copy(idx_hbm.at[pl.ds(base, _CH)], idx_v.at[0])
        pltpu.sync_copy(idx_hbm.at[pl.ds(base + _CH, _CH)], idx_v.at[1])
        c0 = pltpu.async_copy(emb_hbm.at[idx_v.at[0]],
                              rows_v.at[pl.ds(0, _CH)], sem)
        c1 = pltpu.async_copy(emb_hbm.at[idx_v.at[1]],
                              rows_v.at[pl.ds(_CH, _CH)], sem)
        c0.wait()
        c1.wait()
        pltpu.sync_copy(rows_v, out_hbm.at[pl.ds(base, _BPW)])

    return gather_k


def kernel(inputs, emb):
    B, T, D = inputs.shape
    xt = jnp.swapaxes(inputs, 1, 2)
    xsq = jnp.sum(inputs ** 2, axis=2)[:, None, :]
    idx2, dsum = _argmin_call(xt, emb, xsq)
    idx = idx2.reshape(_N)
    emb_p = jnp.pad(emb, ((0, 0), (0, _DP - _D)))
    quantized = _make_sc_gather()(emb_p, idx)[:, :_D].reshape(B, T, D)
    e_latent_loss = jnp.sum(dsum) / jnp.float32(_N * _D)
    vq_loss = jnp.float32(0.25) * e_latent_loss
    return (quantized, idx2.reshape(B, T), vq_loss, e_latent_loss,
            jnp.float32(0.0))


# speedup vs baseline: 1.9146x; 1.0224x over previous
import functools

import jax
import jax.numpy as jnp
from jax import lax
from jax.experimental import pallas as pl
from jax.experimental.pallas import tpu as pltpu
from jax.experimental.pallas import tpu_sc as plsc

_N = 8192
_D = 64
_K = 8192
_TM = 512
_KC = 2048
_GT = _N // _TM
_KS = _K // _KC

_NW = 32
_BPW = _N // _NW
_CH = 128
_DP = 128


def _argmin_body(x_ref, e_ref, xsq_ref, idx_ref, dsum_ref, ep_ref):
    xb = (x_ref[0] * 2.0).astype(jnp.bfloat16)
    xsq = xsq_ref[0]

    @pl.when(pl.program_id(0) == 0)
    def _():
        ep_ref[:, 0:_D] = e_ref[...]

    H = 16
    gbest = None
    gidx = None
    for j in range(_KS):
        e = e_ref[pl.ds(j * _KC, _KC), :]
        esq = jnp.sum(e * e, axis=1, keepdims=True)
        mm = lax.dot_general(e, xb, (((1,), (0,)), ((), ())),
                             preferred_element_type=jnp.float32)
        bval = (xsq + esq[0:H]) - mm[0:H, :]
        bslab = jnp.zeros((H, _TM), jnp.int32)
        for c in range(1, _KC // H):
            d = (xsq + esq[c * H:(c + 1) * H]) - mm[c * H:(c + 1) * H, :]
            better = d < bval
            bval = jnp.where(better, d, bval)
            bslab = jnp.where(better, jnp.int32(c), bslab)
        m = jnp.min(bval, axis=0, keepdims=True)
        kfull = bslab * H + lax.broadcasted_iota(jnp.int32, (H, _TM), 0)
        cand = jnp.min(jnp.where(bval == m, kfull, _K), axis=0,
                       keepdims=True) + j * _KC
        if j == 0:
            gbest = m.astype(jnp.bfloat16).astype(jnp.float32)
            gidx = cand
        else:
            better = m < gbest
            gidx = jnp.where(better, cand, gidx)
            gbest = jnp.where(better, m, gbest).astype(
                jnp.bfloat16).astype(jnp.float32)
    idx_ref[...] = gidx[None]
    dsum_ref[...] = jnp.sum(gbest)[None, None, None]


_argmin_call = pl.pallas_call(
    _argmin_body,
    grid=(_GT,),
    in_specs=[
        pl.BlockSpec((1, _D, _TM), lambda i: (i, 0, 0)),
        pl.BlockSpec((_K, _D), lambda i: (0, 0)),
        pl.BlockSpec((1, 1, _TM), lambda i: (i, 0, 0)),
    ],
    out_specs=[
        pl.BlockSpec((1, 1, _TM), lambda i: (i, 0, 0)),
        pl.BlockSpec((1, 1, 1), lambda i: (i, 0, 0)),
        pl.BlockSpec((_K, _DP), lambda i: (0, 0)),
    ],
    out_shape=[
        jax.ShapeDtypeStruct((_GT, 1, _TM), jnp.int32),
        jax.ShapeDtypeStruct((_GT, 1, 1), jnp.float32),
        jax.ShapeDtypeStruct((_K, _DP), jnp.float32),
    ],
    compiler_params=pltpu.CompilerParams(
        dimension_semantics=("arbitrary",)),
)


@functools.cache
def _make_sc_gather():
    mesh = plsc.VectorSubcoreMesh(core_axis_name="c", subcore_axis_name="s")

    @functools.partial(
        pl.kernel,
        mesh=mesh,
        out_type=jax.ShapeDtypeStruct((_N, _DP), jnp.float32),
        scratch_types=[
            pltpu.VMEM((2, _CH), jnp.int32),
            pltpu.VMEM((_BPW, _DP), jnp.float32),
            pltpu.SemaphoreType.DMA,
        ],
    )
    def gather_k(emb_hbm, idx_hbm, out_hbm, idx_v, rows_v, sem):
        wid = lax.axis_index("s") * 2 + lax.axis_index("c")
        base = wid * _BPW
        pltpu.sync_copy(idx_hbm.at[pl.ds(base, _CH)], idx_v.at[0])
        pltpu.sync_copy(idx_hbm.at[pl.ds(base + _CH, _CH)], idx_v.at[1])
        c0 = pltpu.async_copy(emb_hbm.at[idx_v.at[0]],
                              rows_v.at[pl.ds(0, _CH)], sem)
        c1 = pltpu.async_copy(emb_hbm.at[idx_v.at[1]],
                              rows_v.at[pl.ds(_CH, _CH)], sem)
        c0.wait()
        c1.wait()
        pltpu.sync_copy(rows_v, out_hbm.at[pl.ds(base, _BPW)])

    return gather_k


def kernel(inputs, emb):
    B, T, D = inputs.shape
    xt = jnp.swapaxes(inputs, 1, 2)
    xsq = jnp.sum(inputs ** 2, axis=2)[:, None, :]
    idx2, dsum, emb_p = _argmin_call(xt, emb, xsq)
    idx = idx2.reshape(_N)
    quantized = _make_sc_gather()(emb_p, idx)[:, :_D].reshape(B, T, D)
    e_latent_loss = jnp.sum(dsum) / jnp.float32(_N * _D)
    vq_loss = jnp.float32(0.25) * e_latent_loss
    return (quantized, idx2.reshape(B, T), vq_loss, e_latent_loss,
            jnp.float32(0.0))


# speedup vs baseline: 1.9224x; 1.0041x over previous
import functools

import jax
import jax.numpy as jnp
from jax import lax
from jax.experimental import pallas as pl
from jax.experimental.pallas import tpu as pltpu
from jax.experimental.pallas import tpu_sc as plsc

_N = 8192
_D = 64
_K = 8192
_TM = 512
_KC = 2048
_GT = _N // _TM
_KS = _K // _KC

_NW = 32
_BPW = _N // _NW
_CH = 128
_DP = 128


def _argmin_body(x_ref, e_ref, xsq_ref, idx_ref, dsum_ref, ep_ref):

    @pl.when(pl.program_id(0) == 0)
    def _():
        ep_ref[:, 0:_D] = e_ref[...]

    H = 16
    for b2 in range(2):
      xb = (x_ref[b2] * 2.0).astype(jnp.bfloat16)
      xsq = xsq_ref[b2]
      gbest = None
      gidx = None
      for j in range(_KS):
        e = e_ref[pl.ds(j * _KC, _KC), :]
        esq = jnp.sum(e * e, axis=1, keepdims=True)
        mm = lax.dot_general(e, xb, (((1,), (0,)), ((), ())),
                             preferred_element_type=jnp.float32)
        bval = (xsq + esq[0:H]) - mm[0:H, :]
        bslab = jnp.zeros((H, _TM), jnp.int32)
        for c in range(1, _KC // H):
            d = (xsq + esq[c * H:(c + 1) * H]) - mm[c * H:(c + 1) * H, :]
            better = d < bval
            bval = jnp.where(better, d, bval)
            bslab = jnp.where(better, jnp.int32(c), bslab)
        m = jnp.min(bval, axis=0, keepdims=True)
        kfull = bslab * H + lax.broadcasted_iota(jnp.int32, (H, _TM), 0)
        cand = jnp.min(jnp.where(bval == m, kfull, _K), axis=0,
                       keepdims=True) + j * _KC
        if j == 0:
            gbest = m.astype(jnp.bfloat16).astype(jnp.float32)
            gidx = cand
        else:
            better = m < gbest
            gidx = jnp.where(better, cand, gidx)
            gbest = jnp.where(better, m, gbest).astype(
                jnp.bfloat16).astype(jnp.float32)
      idx_ref[b2] = gidx
      dsum_ref[b2] = jnp.sum(gbest)[None, None]


_argmin_call = pl.pallas_call(
    _argmin_body,
    grid=(_GT // 2,),
    in_specs=[
        pl.BlockSpec((2, _D, _TM), lambda i: (i, 0, 0)),
        pl.BlockSpec((_K, _D), lambda i: (0, 0)),
        pl.BlockSpec((2, 1, _TM), lambda i: (i, 0, 0)),
    ],
    out_specs=[
        pl.BlockSpec((2, 1, _TM), lambda i: (i, 0, 0)),
        pl.BlockSpec((2, 1, 1), lambda i: (i, 0, 0)),
        pl.BlockSpec((_K, _DP), lambda i: (0, 0)),
    ],
    out_shape=[
        jax.ShapeDtypeStruct((_GT, 1, _TM), jnp.int32),
        jax.ShapeDtypeStruct((_GT, 1, 1), jnp.float32),
        jax.ShapeDtypeStruct((_K, _DP), jnp.float32),
    ],
    compiler_params=pltpu.CompilerParams(
        dimension_semantics=("arbitrary",)),
)


@functools.cache
def _make_sc_gather():
    mesh = plsc.VectorSubcoreMesh(core_axis_name="c", subcore_axis_name="s")

    @functools.partial(
        pl.kernel,
        mesh=mesh,
        out_type=jax.ShapeDtypeStruct((_N, _DP), jnp.float32),
        scratch_types=[
            pltpu.VMEM((2, _CH), jnp.int32),
            pltpu.VMEM((_BPW, _DP), jnp.float32),
            pltpu.SemaphoreType.DMA,
        ],
    )
    def gather_k(emb_hbm, idx_hbm, out_hbm, idx_v, rows_v, sem):
        wid = lax.axis_index("s") * 2 + lax.axis_index("c")
        base = wid * _BPW
        pltpu.sync_copy(idx_hbm.at[pl.ds(base, _CH)], idx_v.at[0])
        pltpu.sync_copy(idx_hbm.at[pl.ds(base + _CH, _CH)], idx_v.at[1])
        c0 = pltpu.async_copy(emb_hbm.at[idx_v.at[0]],
                              rows_v.at[pl.ds(0, _CH)], sem)
        c1 = pltpu.async_copy(emb_hbm.at[idx_v.at[1]],
                              rows_v.at[pl.ds(_CH, _CH)], sem)
        c0.wait()
        c1.wait()
        pltpu.sync_copy(rows_v, out_hbm.at[pl.ds(base, _BPW)])

    return gather_k


def kernel(inputs, emb):
    B, T, D = inputs.shape
    xt = jnp.swapaxes(inputs, 1, 2)
    xsq = jnp.sum(inputs ** 2, axis=2)[:, None, :]
    idx2, dsum, emb_p = _argmin_call(xt, emb, xsq)
    idx = idx2.reshape(_N)
    quantized = _make_sc_gather()(emb_p, idx)[:, :_D].reshape(B, T, D)
    e_latent_loss = jnp.sum(dsum) / jnp.float32(_N * _D)
    vq_loss = jnp.float32(0.25) * e_latent_loss
    return (quantized, idx2.reshape(B, T), vq_loss, e_latent_loss,
            jnp.float32(0.0))
